# baseline (speedup 1.0000x reference)
import functools

import jax
import jax.numpy as jnp
from jax import lax
from jax.experimental import pallas as pl
from jax.experimental.pallas import tpu as pltpu
from jax.experimental.pallas import tpu_sc as plsc

N = 10000
H = 128
E = 640000
P = 200000
DP = 26
EPS = 1e-5

NC = 2
NS = 16
LANES = 16
NW = NC * NS
K = 128

N_PAD = 10112
ROWS_SUB = N_PAD // NS
PAD_NODE = 10008
CE = 160
E_PAD = NW * CE * K
DN = 10240
DSUB = DN // NS
CP = 50
P_PAD = NW * CP * K
K2 = 64
BP = 1024
GP = P_PAD // BP

_f32 = jnp.float32


def _sc_mesh():
    return plsc.VectorSubcoreMesh(core_axis_name="c", subcore_axis_name="s",
                                  num_cores=NC, num_subcores=NS)


def _make_sc_deg(interpret=False):
    @functools.partial(
        pl.kernel, mesh=_sc_mesh(), interpret=interpret,
        out_type=jax.ShapeDtypeStruct((NC, DN), _f32),
        scratch_types=[
            pltpu.VMEM((CE, K), jnp.int32),
            pltpu.VMEM((K,), _f32),
            pltpu.VMEM((DSUB,), _f32),
            pltpu.VMEM_SHARED((DN,), _f32),
            pltpu.SemaphoreType.DMA,
        ],
    )
    def deg_kernel(dst_hbm, out_hbm, didx_all, ones_v, zbuf, dacc, sem):
        c = lax.axis_index("c")
        s = lax.axis_index("s")
        wid = s * NC + c
        one16 = jnp.ones((LANES,), _f32)
        zero16 = jnp.zeros((LANES,), _f32)
        for j in range(K // LANES):
            ones_v[pl.ds(j * LANES, LANES)] = one16
        for j in range(DSUB // LANES):
            zbuf[pl.ds(j * LANES, LANES)] = zero16
        pltpu.sync_copy(zbuf, dacc.at[pl.ds(s * DSUB, DSUB)])
        pltpu.sync_copy(dst_hbm.at[wid], didx_all)
        plsc.subcore_barrier()

        FIRE = 8
        assert CE % FIRE == 0

        def body(g, carry):
            for j in range(FIRE):
                pltpu.async_copy(ones_v, dacc.at[didx_all.at[g * FIRE + j]],
                                 sem, add=True)
            for j in range(FIRE):
                pltpu.make_async_copy(
                    ones_v, dacc.at[didx_all.at[g * FIRE + j]], sem).wait()
            return carry

        lax.fori_loop(0, CE // FIRE, body, 0)
        plsc.subcore_barrier()
        pltpu.sync_copy(dacc.at[pl.ds(s * DSUB, DSUB)],
                        out_hbm.at[c, pl.ds(s * DSUB, DSUB)])

    return deg_kernel


def _make_sc_msg(interpret=False):
    @functools.partial(
        pl.kernel, mesh=_sc_mesh(), interpret=interpret,
        out_type=jax.ShapeDtypeStruct((NC, N_PAD, H), _f32),
        scratch_types=[
            pltpu.VMEM((4, K2), jnp.int32),
            pltpu.VMEM((4, K2), jnp.int32),
            pltpu.VMEM((4, K2, H), _f32),
            pltpu.VMEM_SHARED((N_PAD, H), _f32),
            pltpu.SemaphoreType.DMA,
            pltpu.SemaphoreType.DMA,
            pltpu.SemaphoreType.DMA,
            pltpu.SemaphoreType.DMA,
            pltpu.SemaphoreType.DMA,
            pltpu.SemaphoreType.DMA,
            pltpu.SemaphoreType.DMA,
            pltpu.SemaphoreType.DMA,
        ],
    )
    def msg_kernel(hp_hbm, src_hbm, dst_hbm, out_hbm, sidx, didx,
                   rows, acc, g0, g1, g2, g3, s0, s1, s2, s3):
        c = lax.axis_index("c")
        s = lax.axis_index("s")
        wid = s * NC + c
        row0 = s * ROWS_SUB

        @pl.when(c == 0)
        def _():
            pltpu.sync_copy(hp_hbm.at[pl.ds(row0, ROWS_SUB)],
                            acc.at[pl.ds(row0, ROWS_SUB)])

        @pl.when(c != 0)
        def _():
            zero16 = jnp.zeros((LANES,), _f32)

            def zb(i, carry):
                for j in range(H // LANES):
                    rows[0, i, pl.ds(j * LANES, LANES)] = zero16
                return carry

            lax.fori_loop(0, K2, zb, 0)
            nfull = ROWS_SUB // K2
            rem = ROWS_SUB - nfull * K2
            for t in range(nfull):
                pltpu.sync_copy(rows.at[0], acc.at[pl.ds(row0 + t * K2, K2)])
            pltpu.sync_copy(rows.at[0, pl.ds(0, rem)],
                            acc.at[pl.ds(row0 + nfull * K2, rem)])

        plsc.subcore_barrier()

        gsems = (g0, g1, g2, g3)
        ssems = (s0, s1, s2, s3)
        NG = E_PAD // (NW * 4 * K2)

        def body(g, carry):
            j = wid * NG + g
            pltpu.sync_copy(src_hbm.at[j], sidx)
            pltpu.sync_copy(dst_hbm.at[j], didx)
            ds = [pltpu.async_copy(hp_hbm.at[sidx.at[u]], rows.at[u],
                                   gsems[u]) for u in range(4)]
            ws = []
            for u in range(4):
                ds[u].wait()
                ws.append(pltpu.async_copy(rows.at[u], acc.at[didx.at[u]],
                                           ssems[u], add=True))
            for u in range(4):
                ws[u].wait()
            return carry

        lax.fori_loop(0, NG, body, 0)
        plsc.subcore_barrier()
        pltpu.sync_copy(acc.at[pl.ds(row0, ROWS_SUB)],
                        out_hbm.at[c, pl.ds(row0, ROWS_SUB)])

    return msg_kernel


def _make_sc_pair(interpret=False):
    @functools.partial(
        pl.kernel, mesh=_sc_mesh(), interpret=interpret,
        out_type=[jax.ShapeDtypeStruct((P_PAD, H), _f32),
                  jax.ShapeDtypeStruct((P_PAD, H), _f32)],
        scratch_types=[
            pltpu.VMEM((CP, K), jnp.int32),
            pltpu.VMEM((CP, K), jnp.int32),
            pltpu.VMEM((2, K, H), _f32),
            pltpu.VMEM((2, K, H), _f32),
            pltpu.SemaphoreType.DMA,
            pltpu.SemaphoreType.DMA,
            pltpu.SemaphoreType.DMA,
            pltpu.SemaphoreType.DMA,
            pltpu.SemaphoreType.DMA,
            pltpu.SemaphoreType.DMA,
            pltpu.SemaphoreType.DMA,
            pltpu.SemaphoreType.DMA,
        ],
    )
    def pair_kernel(g0_hbm, g1_hbm, i0_hbm, i1_hbm, a0_hbm, a1_hbm,
                    i0_all, i1_all, r0, r1,
                    g0s0, g0s1, g1s0, g1s1, w0s0, w0s1, w1s0, w1s1):
        c = lax.axis_index("c")
        s = lax.axis_index("s")
        wid = s * NC + c
        pltpu.sync_copy(i0_hbm.at[wid], i0_all)
        pltpu.sync_copy(i1_hbm.at[wid], i1_all)
        rbase = wid * (CP * K)

        def body(g, carry):
            c0 = g * 2
            c1 = g * 2 + 1
            d00 = pltpu.async_copy(g0_hbm.at[i0_all.at[c0]], r0.at[0], g0s0)
            d01 = pltpu.async_copy(g0_hbm.at[i0_all.at[c1]], r0.at[1], g0s1)
            d10 = pltpu.async_copy(g1_hbm.at[i1_all.at[c0]], r1.at[0], g1s0)
            d11 = pltpu.async_copy(g1_hbm.at[i1_all.at[c1]], r1.at[1], g1s1)
            d00.wait()
            w00 = pltpu.async_copy(r0.at[0],
                                   a0_hbm.at[pl.ds(rbase + c0 * K, K)], w0s0)
            d01.wait()
            w01 = pltpu.async_copy(r0.at[1],
                                   a0_hbm.at[pl.ds(rbase + c1 * K, K)], w0s1)
            d10.wait()
            w10 = pltpu.async_copy(r1.at[0],
                                   a1_hbm.at[pl.ds(rbase + c0 * K, K)], w1s0)
            d11.wait()
            w11 = pltpu.async_copy(r1.at[1],
                                   a1_hbm.at[pl.ds(rbase + c1 * K, K)], w1s1)
            w00.wait()
            w01.wait()
            w10.wait()
            w11.wait()
            return carry

        lax.fori_loop(0, CP // 2, body, 0)

    return pair_kernel


_make_sc_deg = functools.lru_cache(maxsize=None)(_make_sc_deg)
_make_sc_msg = functools.lru_cache(maxsize=None)(_make_sc_msg)
_make_sc_pair = functools.lru_cache(maxsize=None)(_make_sc_pair)


def _sc_deg(dstp):
    return _make_sc_deg()(dstp)


def _sc_msg(hp, srcp, dstp):
    return _make_sc_msg()(hp, srcp, dstp)


def _sc_pair(G0, G1, i0, i1):
    return _make_sc_pair()(G0, G1, i0, i1)


def _tc_embed(x, emb_W, emb_b, emb_g, emb_be, degp3, W0, interpret=False):
    def body(x_ref, w_ref, b_ref, g_ref, be_ref, dp_ref, w0_ref,
             hp_ref, dinv_ref):
        y = jnp.dot(x_ref[...], w_ref[...],
                    preferred_element_type=_f32) + b_ref[...]
        m = jnp.mean(y, axis=0, keepdims=True)
        yc = y - m
        v = jnp.mean(yc * yc, axis=0, keepdims=True)
        h = jnp.maximum(yc * lax.rsqrt(v + EPS) * g_ref[...] + be_ref[...],
                        0.0)
        deg = dp_ref[0] + dp_ref[1]
        dinv = lax.rsqrt(deg[:N] + 1.0)
        dinv_ref[...] = dinv
        hp_ref[:N] = dinv * jnp.dot(h, w0_ref[...], preferred_element_type=_f32)
        hp_ref[N:] = jnp.zeros((N_PAD - N, H), _f32)

    return pl.pallas_call(
        body, interpret=interpret,
        out_shape=[jax.ShapeDtypeStruct((N_PAD, H), _f32),
                   jax.ShapeDtypeStruct((N, 1), _f32)],
    )(x, emb_W, emb_b, emb_g, emb_be, degp3, W0)


def _tc_layer(Ppart, dinv, b, g, be, Wn, interpret=False):
    def body(p_ref, dinv_ref, b_ref, g_ref, be_ref, w_ref, out_ref):
        y = dinv_ref[...] * (p_ref[0, :N] + p_ref[1, :N]) + b_ref[...]
        m = jnp.mean(y, axis=0, keepdims=True)
        yc = y - m
        v = jnp.mean(yc * yc, axis=0, keepdims=True)
        h = jnp.maximum(yc * lax.rsqrt(v + EPS) * g_ref[...] + be_ref[...],
                        0.0)
        out_ref[:N] = dinv_ref[...] * jnp.dot(h, w_ref[...],
                                              preferred_element_type=_f32)
        out_ref[N:] = jnp.zeros((N_PAD - N, H), _f32)

    return pl.pallas_call(
        body, interpret=interpret,
        out_shape=jax.ShapeDtypeStruct((N_PAD, H), _f32),
    )(Ppart, dinv, b, g, be, Wn)


def _tc_final(Ppart, dinv, b, g, be, W1a, W1b, interpret=False):
    def body(p_ref, dinv_ref, b_ref, g_ref, be_ref, wa_ref, wb_ref,
             g0_ref, g1_ref):
        y = dinv_ref[...] * (p_ref[0, :N] + p_ref[1, :N]) + b_ref[...]
        m = jnp.mean(y, axis=0, keepdims=True)
        yc = y - m
        v = jnp.mean(yc * yc, axis=0, keepdims=True)
        h = jnp.maximum(yc * lax.rsqrt(v + EPS) * g_ref[...] + be_ref[...],
                        0.0)
        g0_ref[...] = jnp.dot(h, wa_ref[...], preferred_element_type=_f32)
        g1_ref[...] = jnp.dot(h, wb_ref[...], preferred_element_type=_f32)

    return pl.pallas_call(
        body, interpret=interpret,
        out_shape=[jax.ShapeDtypeStruct((N, H), _f32),
                   jax.ShapeDtypeStruct((N, H), _f32)],
    )(Ppart, dinv, b, g, be, W1a, W1b)


def _tc_passA(A0, A1, pfp, W1c, b1, interpret=False):
    def body(a0, a1, pf, wc, b, y_ref, s1_ref, s2_ref):
        i = pl.program_id(0)
        y = a0[...] + a1[...] + jnp.dot(pf[...], wc[...],
                                        preferred_element_type=_f32) + b[...]
        y_ref[...] = y
        rid = i * BP + lax.broadcasted_iota(jnp.int32, (BP, 1), 0)
        ym = jnp.where(rid < P, y, 0.0)
        ps1 = jnp.sum(ym, axis=0, keepdims=True)
        ps2 = jnp.sum(ym * ym, axis=0, keepdims=True)

        @pl.when(i == 0)
        def _():
            s1_ref[...] = ps1
            s2_ref[...] = ps2

        @pl.when(i != 0)
        def _():
            s1_ref[...] = s1_ref[...] + ps1
            s2_ref[...] = s2_ref[...] + ps2

    return pl.pallas_call(
        body, interpret=interpret, grid=(GP,),
        in_specs=[pl.BlockSpec((BP, H), lambda i: (i, 0)),
                  pl.BlockSpec((BP, H), lambda i: (i, 0)),
                  pl.BlockSpec((BP, DP), lambda i: (i, 0)),
                  pl.BlockSpec((DP, H), lambda i: (0, 0)),
                  pl.BlockSpec((1, H), lambda i: (0, 0))],
        out_specs=[pl.BlockSpec((BP, H), lambda i: (i, 0)),
                   pl.BlockSpec((1, H), lambda i: (0, 0)),
                   pl.BlockSpec((1, H), lambda i: (0, 0))],
        out_shape=[jax.ShapeDtypeStruct((P_PAD, H), _f32),
                   jax.ShapeDtypeStruct((1, H), _f32),
                   jax.ShapeDtypeStruct((1, H), _f32)],
    )(A0, A1, pfp, W1c, b1)


def _tc_passB(y, s1, s2, g1, be1, W2, b2, interpret=False):
    H2 = H // 2

    def body(y_ref, s1r, s2r, gr, ber, wr, br, u_ref, t1_ref, t2_ref):
        i = pl.program_id(0)
        m = s1r[...] * (1.0 / P)
        var = s2r[...] * (1.0 / P) - m * m
        z = jnp.maximum((y_ref[...] - m) * lax.rsqrt(var + EPS) * gr[...]
                        + ber[...], 0.0)
        u = jnp.dot(z, wr[...], preferred_element_type=_f32) + br[...]
        u_ref[...] = u
        rid = i * BP + lax.broadcasted_iota(jnp.int32, (BP, 1), 0)
        um = jnp.where(rid < P, u, 0.0)
        ps1 = jnp.sum(um, axis=0, keepdims=True)
        ps2 = jnp.sum(um * um, axis=0, keepdims=True)

        @pl.when(i == 0)
        def _():
            t1_ref[...] = ps1
            t2_ref[...] = ps2

        @pl.when(i != 0)
        def _():
            t1_ref[...] = t1_ref[...] + ps1
            t2_ref[...] = t2_ref[...] + ps2

    return pl.pallas_call(
        body, interpret=interpret, grid=(GP,),
        in_specs=[pl.BlockSpec((BP, H), lambda i: (i, 0)),
                  pl.BlockSpec((1, H), lambda i: (0, 0)),
                  pl.BlockSpec((1, H), lambda i: (0, 0)),
                  pl.BlockSpec((1, H), lambda i: (0, 0)),
                  pl.BlockSpec((1, H), lambda i: (0, 0)),
                  pl.BlockSpec((H, H2), lambda i: (0, 0)),
                  pl.BlockSpec((1, H2), lambda i: (0, 0))],
        out_specs=[pl.BlockSpec((BP, H2), lambda i: (i, 0)),
                   pl.BlockSpec((1, H2), lambda i: (0, 0)),
                   pl.BlockSpec((1, H2), lambda i: (0, 0))],
        out_shape=[jax.ShapeDtypeStruct((P_PAD, H2), _f32),
                   jax.ShapeDtypeStruct((1, H2), _f32),
                   jax.ShapeDtypeStruct((1, H2), _f32)],
    )(y, s1, s2, g1, be1, W2, b2)


def _tc_passC(u, t1, t2, g2, be2, W3, b3, interpret=False):
    H2 = H // 2

    def body(u_ref, t1r, t2r, gr, ber, wr, br, o_ref):
        m = t1r[...] * (1.0 / P)
        var = t2r[...] * (1.0 / P) - m * m
        z = jnp.maximum((u_ref[...] - m) * lax.rsqrt(var + EPS) * gr[...]
                        + ber[...], 0.0)
        o_ref[...] = jnp.dot(z, wr[...], preferred_element_type=_f32) + br[...]

    return pl.pallas_call(
        body, interpret=interpret, grid=(GP,),
        in_specs=[pl.BlockSpec((BP, H2), lambda i: (i, 0)),
                  pl.BlockSpec((1, H2), lambda i: (0, 0)),
                  pl.BlockSpec((1, H2), lambda i: (0, 0)),
                  pl.BlockSpec((1, H2), lambda i: (0, 0)),
                  pl.BlockSpec((1, H2), lambda i: (0, 0)),
                  pl.BlockSpec((H2, 1), lambda i: (0, 0)),
                  pl.BlockSpec((1, 1), lambda i: (0, 0))],
        out_specs=pl.BlockSpec((BP, 1), lambda i: (i, 0)),
        out_shape=jax.ShapeDtypeStruct((P_PAD, 1), _f32),
    )(u, t1, t2, g2, be2, W3, b3)


def kernel(x, edge_index, batch, pair_indices, pair_features,
           emb_W, emb_b, emb_g, emb_be,
           conv_W, conv_b, bn_g, bn_be,
           mlp1_W, mlp1_b, mlp1_g, mlp1_be,
           mlp2_W, mlp2_b, mlp2_g, mlp2_be,
           mlp3_W, mlp3_b):
    f32 = _f32
    epad = N + (jnp.arange(E_PAD - E, dtype=jnp.int32) % (N_PAD - N))
    srcp = jnp.concatenate([edge_index[0].astype(jnp.int32),
                            epad]).reshape(NW, CE, K)
    dstp = jnp.concatenate([edge_index[1].astype(jnp.int32),
                            epad]).reshape(NW, CE, K)
    ppad = jnp.arange(P_PAD - P, dtype=jnp.int32) % N
    i0 = jnp.concatenate([pair_indices[:, 0].astype(jnp.int32),
                          ppad]).reshape(NW, CP, K)
    i1 = jnp.concatenate([pair_indices[:, 1].astype(jnp.int32),
                          ppad]).reshape(NW, CP, K)
    pfp = jnp.concatenate(
        [pair_features.astype(f32), jnp.zeros((P_PAD - P, DP), f32)], axis=0)

    def r(a):
        return a.astype(f32).reshape(1, -1)

    degp = _sc_deg(dstp)
    degp3 = degp.reshape(NC, DN, 1)
    hp, dinv = _tc_embed(x.astype(f32), emb_W.astype(f32), r(emb_b),
                         r(emb_g), r(emb_be), degp3, conv_W[0].astype(f32))
    srcp1 = srcp.reshape(-1, 4, K2)
    dstp1 = dstp.reshape(-1, 4, K2)
    G0 = G1 = None
    for l in range(4):
        Ppart = _sc_msg(hp, srcp1, dstp1)
        if l < 3:
            hp = _tc_layer(Ppart, dinv, r(conv_b[l]), r(bn_g[l]),
                           r(bn_be[l]), conv_W[l + 1].astype(f32))
        else:
            G0, G1 = _tc_final(Ppart, dinv, r(conv_b[l]), r(bn_g[l]),
                               r(bn_be[l]), mlp1_W[:H].astype(f32),
                               mlp1_W[H:2 * H].astype(f32))
    A0, A1 = _sc_pair(G0, G1, i0, i1)
    y, s1, s2 = _tc_passA(A0, A1, pfp, mlp1_W[2 * H:].astype(f32), r(mlp1_b))
    u, t1, t2 = _tc_passB(y, s1, s2, r(mlp1_g), r(mlp1_be),
                          mlp2_W.astype(f32), r(mlp2_b))
    out = _tc_passC(u, t1, t2, r(mlp2_g), r(mlp2_be),
                    mlp3_W.astype(f32), mlp3_b.astype(f32).reshape(1, 1))
    return out[:P]


# speedup vs baseline: 1.0364x; 1.0364x over previous
import functools

import jax
import jax.numpy as jnp
from jax import lax
from jax.experimental import pallas as pl
from jax.experimental.pallas import tpu as pltpu
from jax.experimental.pallas import tpu_sc as plsc

N = 10000
H = 128
E = 640000
P = 200000
DP = 26
EPS = 1e-5

NC = 2
NS = 16
LANES = 16
NW = NC * NS
K = 128

N_PAD = 10112
ROWS_SUB = N_PAD // NS
PAD_NODE = 10008
CE = 160
E_PAD = NW * CE * K
DN = 10240
DSUB = DN // NS
CP = 50
P_PAD = NW * CP * K
K2 = 64
BP = 1024
GP = P_PAD // BP

_f32 = jnp.float32


def _sc_mesh():
    return plsc.VectorSubcoreMesh(core_axis_name="c", subcore_axis_name="s",
                                  num_cores=NC, num_subcores=NS)


def _make_sc_deg(interpret=False):
    @functools.partial(
        pl.kernel, mesh=_sc_mesh(), interpret=interpret,
        out_type=jax.ShapeDtypeStruct((NC, DN), _f32),
        scratch_types=[
            pltpu.VMEM((CE, K), jnp.int32),
            pltpu.VMEM((K,), _f32),
            pltpu.VMEM((DSUB,), _f32),
            pltpu.VMEM_SHARED((DN,), _f32),
            pltpu.SemaphoreType.DMA,
        ],
    )
    def deg_kernel(dst_hbm, out_hbm, didx_all, ones_v, zbuf, dacc, sem):
        c = lax.axis_index("c")
        s = lax.axis_index("s")
        wid = s * NC + c
        one16 = jnp.ones((LANES,), _f32)
        zero16 = jnp.zeros((LANES,), _f32)
        for j in range(K // LANES):
            ones_v[pl.ds(j * LANES, LANES)] = one16
        for j in range(DSUB // LANES):
            zbuf[pl.ds(j * LANES, LANES)] = zero16
        pltpu.sync_copy(zbuf, dacc.at[pl.ds(s * DSUB, DSUB)])
        pltpu.sync_copy(dst_hbm.at[wid], didx_all)
        plsc.subcore_barrier()

        FIRE = 8
        assert CE % FIRE == 0

        def body(g, carry):
            for j in range(FIRE):
                pltpu.async_copy(ones_v, dacc.at[didx_all.at[g * FIRE + j]],
                                 sem, add=True)
            for j in range(FIRE):
                pltpu.make_async_copy(
                    ones_v, dacc.at[didx_all.at[g * FIRE + j]], sem).wait()
            return carry

        lax.fori_loop(0, CE // FIRE, body, 0)
        plsc.subcore_barrier()
        pltpu.sync_copy(dacc.at[pl.ds(s * DSUB, DSUB)],
                        out_hbm.at[c, pl.ds(s * DSUB, DSUB)])

    return deg_kernel


def _make_sc_msg(interpret=False):
    @functools.partial(
        pl.kernel, mesh=_sc_mesh(), interpret=interpret,
        out_type=jax.ShapeDtypeStruct((NC, N_PAD, H), _f32),
        scratch_types=[
            pltpu.VMEM((2, K), jnp.int32),
            pltpu.VMEM((2, K), jnp.int32),
            pltpu.VMEM((2, K, H), _f32),
            pltpu.VMEM_SHARED((N_PAD, H), _f32),
            pltpu.SemaphoreType.DMA,
            pltpu.SemaphoreType.DMA,
            pltpu.SemaphoreType.DMA,
            pltpu.SemaphoreType.DMA,
        ],
    )
    def msg_kernel(hp_hbm, src_hbm, dst_hbm, out_hbm, sidx, didx,
                   rows, acc, g0, g1, s0, s1):
        c = lax.axis_index("c")
        s = lax.axis_index("s")
        wid = s * NC + c
        row0 = s * ROWS_SUB

        @pl.when(c == 0)
        def _():
            pltpu.sync_copy(hp_hbm.at[pl.ds(row0, ROWS_SUB)],
                            acc.at[pl.ds(row0, ROWS_SUB)])

        @pl.when(c != 0)
        def _():
            zero16 = jnp.zeros((LANES,), _f32)

            def zb(i, carry):
                for j in range(H // LANES):
                    rows[0, i, pl.ds(j * LANES, LANES)] = zero16
                return carry

            lax.fori_loop(0, K, zb, 0)
            nfull = ROWS_SUB // K
            rem = ROWS_SUB - nfull * K
            for t in range(nfull):
                pltpu.sync_copy(rows.at[0], acc.at[pl.ds(row0 + t * K, K)])
            pltpu.sync_copy(rows.at[0, pl.ds(0, rem)],
                            acc.at[pl.ds(row0 + nfull * K, rem)])

        plsc.subcore_barrier()

        gsems = (g0, g1)
        ssems = (s0, s1)
        NG = E_PAD // (NW * 2 * K)

        def body(g, carry):
            j = wid * NG + g
            pltpu.sync_copy(src_hbm.at[j], sidx)
            pltpu.sync_copy(dst_hbm.at[j], didx)
            ds = [pltpu.async_copy(hp_hbm.at[sidx.at[u]], rows.at[u],
                                   gsems[u]) for u in range(2)]
            ws = []
            for u in range(2):
                ds[u].wait()
                ws.append(pltpu.async_copy(rows.at[u], acc.at[didx.at[u]],
                                           ssems[u], add=True))
            for u in range(2):
                ws[u].wait()
            return carry

        lax.fori_loop(0, NG, body, 0)
        plsc.subcore_barrier()
        pltpu.sync_copy(acc.at[pl.ds(row0, ROWS_SUB)],
                        out_hbm.at[c, pl.ds(row0, ROWS_SUB)])

    return msg_kernel


def _make_sc_pair(interpret=False):
    @functools.partial(
        pl.kernel, mesh=_sc_mesh(), interpret=interpret,
        out_type=jax.ShapeDtypeStruct((P_PAD, H), _f32),
        scratch_types=[
            pltpu.VMEM((CP, K), jnp.int32),
            pltpu.VMEM((CP, K), jnp.int32),
            pltpu.VMEM((2, K, H), _f32),
            pltpu.VMEM((2, K, H), _f32),
            pltpu.SemaphoreType.DMA,
            pltpu.SemaphoreType.DMA,
            pltpu.SemaphoreType.DMA,
            pltpu.SemaphoreType.DMA,
            pltpu.SemaphoreType.DMA,
            pltpu.SemaphoreType.DMA,
        ],
    )
    def pair_kernel(g0_hbm, g1_hbm, i0_hbm, i1_hbm, s_hbm,
                    i0_all, i1_all, r0, r1,
                    g0s0, g0s1, g1s0, g1s1, ws0, ws1):
        c = lax.axis_index("c")
        s = lax.axis_index("s")
        wid = s * NC + c
        pltpu.sync_copy(i0_hbm.at[wid], i0_all)
        pltpu.sync_copy(i1_hbm.at[wid], i1_all)
        rbase = wid * (CP * K)

        def vadd(b):
            def ab(i, carry):
                for jj in range(H // LANES):
                    sl = pl.ds(jj * LANES, LANES)
                    r0[b, i, sl] = r0[b, i, sl] + r1[b, i, sl]
                return carry

            lax.fori_loop(0, K, ab, 0)

        def body(g, carry):
            c0 = g * 2
            c1 = g * 2 + 1
            d00 = pltpu.async_copy(g0_hbm.at[i0_all.at[c0]], r0.at[0], g0s0)
            d01 = pltpu.async_copy(g0_hbm.at[i0_all.at[c1]], r0.at[1], g0s1)
            d10 = pltpu.async_copy(g1_hbm.at[i1_all.at[c0]], r1.at[0], g1s0)
            d11 = pltpu.async_copy(g1_hbm.at[i1_all.at[c1]], r1.at[1], g1s1)
            d00.wait()
            d10.wait()
            vadd(0)
            w0 = pltpu.async_copy(r0.at[0],
                                  s_hbm.at[pl.ds(rbase + c0 * K, K)], ws0)
            d01.wait()
            d11.wait()
            vadd(1)
            w1 = pltpu.async_copy(r0.at[1],
                                  s_hbm.at[pl.ds(rbase + c1 * K, K)], ws1)
            w0.wait()
            w1.wait()
            return carry

        lax.fori_loop(0, CP // 2, body, 0)

    return pair_kernel


_make_sc_deg = functools.lru_cache(maxsize=None)(_make_sc_deg)
_make_sc_msg = functools.lru_cache(maxsize=None)(_make_sc_msg)
_make_sc_pair = functools.lru_cache(maxsize=None)(_make_sc_pair)


def _sc_deg(dstp):
    return _make_sc_deg()(dstp)


def _sc_msg(hp, srcp, dstp):
    return _make_sc_msg()(hp, srcp, dstp)


def _sc_pair(G0, G1, i0, i1):
    return _make_sc_pair()(G0, G1, i0, i1)


def _tc_embed(x, emb_W, emb_b, emb_g, emb_be, degp3, W0, interpret=False):
    def body(x_ref, w_ref, b_ref, g_ref, be_ref, dp_ref, w0_ref,
             hp_ref, dinv_ref):
        y = jnp.dot(x_ref[...], w_ref[...],
                    preferred_element_type=_f32) + b_ref[...]
        m = jnp.mean(y, axis=0, keepdims=True)
        yc = y - m
        v = jnp.mean(yc * yc, axis=0, keepdims=True)
        h = jnp.maximum(yc * lax.rsqrt(v + EPS) * g_ref[...] + be_ref[...],
                        0.0)
        deg = dp_ref[0] + dp_ref[1]
        dinv = lax.rsqrt(deg[:N] + 1.0)
        dinv_ref[...] = dinv
        hp_ref[:N] = dinv * jnp.dot(h, w0_ref[...], preferred_element_type=_f32)
        hp_ref[N:] = jnp.zeros((N_PAD - N, H), _f32)

    return pl.pallas_call(
        body, interpret=interpret,
        out_shape=[jax.ShapeDtypeStruct((N_PAD, H), _f32),
                   jax.ShapeDtypeStruct((N, 1), _f32)],
    )(x, emb_W, emb_b, emb_g, emb_be, degp3, W0)


def _tc_layer(Ppart, dinv, b, g, be, Wn, interpret=False):
    def body(p_ref, dinv_ref, b_ref, g_ref, be_ref, w_ref, out_ref):
        y = dinv_ref[...] * (p_ref[0, :N] + p_ref[1, :N]) + b_ref[...]
        m = jnp.mean(y, axis=0, keepdims=True)
        yc = y - m
        v = jnp.mean(yc * yc, axis=0, keepdims=True)
        h = jnp.maximum(yc * lax.rsqrt(v + EPS) * g_ref[...] + be_ref[...],
                        0.0)
        out_ref[:N] = dinv_ref[...] * jnp.dot(h, w_ref[...],
                                              preferred_element_type=_f32)
        out_ref[N:] = jnp.zeros((N_PAD - N, H), _f32)

    return pl.pallas_call(
        body, interpret=interpret,
        out_shape=jax.ShapeDtypeStruct((N_PAD, H), _f32),
    )(Ppart, dinv, b, g, be, Wn)


def _tc_final(Ppart, dinv, b, g, be, W1a, W1b, interpret=False):
    def body(p_ref, dinv_ref, b_ref, g_ref, be_ref, wa_ref, wb_ref,
             g0_ref, g1_ref):
        y = dinv_ref[...] * (p_ref[0, :N] + p_ref[1, :N]) + b_ref[...]
        m = jnp.mean(y, axis=0, keepdims=True)
        yc = y - m
        v = jnp.mean(yc * yc, axis=0, keepdims=True)
        h = jnp.maximum(yc * lax.rsqrt(v + EPS) * g_ref[...] + be_ref[...],
                        0.0)
        g0_ref[...] = jnp.dot(h, wa_ref[...], preferred_element_type=_f32)
        g1_ref[...] = jnp.dot(h, wb_ref[...], preferred_element_type=_f32)

    return pl.pallas_call(
        body, interpret=interpret,
        out_shape=[jax.ShapeDtypeStruct((N, H), _f32),
                   jax.ShapeDtypeStruct((N, H), _f32)],
    )(Ppart, dinv, b, g, be, W1a, W1b)


def _tc_passA(S, pfp, W1c, b1, interpret=False):
    def body(a0, pf, wc, b, y_ref, s1_ref, s2_ref):
        i = pl.program_id(0)
        y = a0[...] + jnp.dot(pf[...], wc[...],
                              preferred_element_type=_f32) + b[...]
        y_ref[...] = y
        rid = i * BP + lax.broadcasted_iota(jnp.int32, (BP, 1), 0)
        ym = jnp.where(rid < P, y, 0.0)
        ps1 = jnp.sum(ym, axis=0, keepdims=True)
        ps2 = jnp.sum(ym * ym, axis=0, keepdims=True)

        @pl.when(i == 0)
        def _():
            s1_ref[...] = ps1
            s2_ref[...] = ps2

        @pl.when(i != 0)
        def _():
            s1_ref[...] = s1_ref[...] + ps1
            s2_ref[...] = s2_ref[...] + ps2

    return pl.pallas_call(
        body, interpret=interpret, grid=(GP,),
        in_specs=[pl.BlockSpec((BP, H), lambda i: (i, 0)),
                  pl.BlockSpec((BP, DP), lambda i: (i, 0)),
                  pl.BlockSpec((DP, H), lambda i: (0, 0)),
                  pl.BlockSpec((1, H), lambda i: (0, 0))],
        out_specs=[pl.BlockSpec((BP, H), lambda i: (i, 0)),
                   pl.BlockSpec((1, H), lambda i: (0, 0)),
                   pl.BlockSpec((1, H), lambda i: (0, 0))],
        out_shape=[jax.ShapeDtypeStruct((P_PAD, H), _f32),
                   jax.ShapeDtypeStruct((1, H), _f32),
                   jax.ShapeDtypeStruct((1, H), _f32)],
    )(S, pfp, W1c, b1)


def _tc_passB(y, s1, s2, g1, be1, W2, b2, interpret=False):
    H2 = H // 2

    def body(y_ref, s1r, s2r, gr, ber, wr, br, u_ref, t1_ref, t2_ref):
        i = pl.program_id(0)
        m = s1r[...] * (1.0 / P)
        var = s2r[...] * (1.0 / P) - m * m
        z = jnp.maximum((y_ref[...] - m) * lax.rsqrt(var + EPS) * gr[...]
                        + ber[...], 0.0)
        u = jnp.dot(z, wr[...], preferred_element_type=_f32) + br[...]
        u_ref[...] = u
        rid = i * BP + lax.broadcasted_iota(jnp.int32, (BP, 1), 0)
        um = jnp.where(rid < P, u, 0.0)
        ps1 = jnp.sum(um, axis=0, keepdims=True)
        ps2 = jnp.sum(um * um, axis=0, keepdims=True)

        @pl.when(i == 0)
        def _():
            t1_ref[...] = ps1
            t2_ref[...] = ps2

        @pl.when(i != 0)
        def _():
            t1_ref[...] = t1_ref[...] + ps1
            t2_ref[...] = t2_ref[...] + ps2

    return pl.pallas_call(
        body, interpret=interpret, grid=(GP,),
        in_specs=[pl.BlockSpec((BP, H), lambda i: (i, 0)),
                  pl.BlockSpec((1, H), lambda i: (0, 0)),
                  pl.BlockSpec((1, H), lambda i: (0, 0)),
                  pl.BlockSpec((1, H), lambda i: (0, 0)),
                  pl.BlockSpec((1, H), lambda i: (0, 0)),
                  pl.BlockSpec((H, H2), lambda i: (0, 0)),
                  pl.BlockSpec((1, H2), lambda i: (0, 0))],
        out_specs=[pl.BlockSpec((BP, H2), lambda i: (i, 0)),
                   pl.BlockSpec((1, H2), lambda i: (0, 0)),
                   pl.BlockSpec((1, H2), lambda i: (0, 0))],
        out_shape=[jax.ShapeDtypeStruct((P_PAD, H2), _f32),
                   jax.ShapeDtypeStruct((1, H2), _f32),
                   jax.ShapeDtypeStruct((1, H2), _f32)],
    )(y, s1, s2, g1, be1, W2, b2)


def _tc_passC(u, t1, t2, g2, be2, W3, b3, interpret=False):
    H2 = H // 2

    def body(u_ref, t1r, t2r, gr, ber, wr, br, o_ref):
        m = t1r[...] * (1.0 / P)
        var = t2r[...] * (1.0 / P) - m * m
        z = jnp.maximum((u_ref[...] - m) * lax.rsqrt(var + EPS) * gr[...]
                        + ber[...], 0.0)
        o_ref[...] = jnp.dot(z, wr[...], preferred_element_type=_f32) + br[...]

    return pl.pallas_call(
        body, interpret=interpret, grid=(GP,),
        in_specs=[pl.BlockSpec((BP, H2), lambda i: (i, 0)),
                  pl.BlockSpec((1, H2), lambda i: (0, 0)),
                  pl.BlockSpec((1, H2), lambda i: (0, 0)),
                  pl.BlockSpec((1, H2), lambda i: (0, 0)),
                  pl.BlockSpec((1, H2), lambda i: (0, 0)),
                  pl.BlockSpec((H2, 1), lambda i: (0, 0)),
                  pl.BlockSpec((1, 1), lambda i: (0, 0))],
        out_specs=pl.BlockSpec((BP, 1), lambda i: (i, 0)),
        out_shape=jax.ShapeDtypeStruct((P_PAD, 1), _f32),
    )(u, t1, t2, g2, be2, W3, b3)


def kernel(x, edge_index, batch, pair_indices, pair_features,
           emb_W, emb_b, emb_g, emb_be,
           conv_W, conv_b, bn_g, bn_be,
           mlp1_W, mlp1_b, mlp1_g, mlp1_be,
           mlp2_W, mlp2_b, mlp2_g, mlp2_be,
           mlp3_W, mlp3_b):
    f32 = _f32
    epad = N + (jnp.arange(E_PAD - E, dtype=jnp.int32) % (N_PAD - N))
    srcp = jnp.concatenate([edge_index[0].astype(jnp.int32),
                            epad]).reshape(NW, CE, K)
    dstp = jnp.concatenate([edge_index[1].astype(jnp.int32),
                            epad]).reshape(NW, CE, K)
    ppad = jnp.arange(P_PAD - P, dtype=jnp.int32) % N
    i0 = jnp.concatenate([pair_indices[:, 0].astype(jnp.int32),
                          ppad]).reshape(NW, CP, K)
    i1 = jnp.concatenate([pair_indices[:, 1].astype(jnp.int32),
                          ppad]).reshape(NW, CP, K)
    pfp = jnp.concatenate(
        [pair_features.astype(f32), jnp.zeros((P_PAD - P, DP), f32)], axis=0)

    def r(a):
        return a.astype(f32).reshape(1, -1)

    degp = _sc_deg(dstp)
    degp3 = degp.reshape(NC, DN, 1)
    hp, dinv = _tc_embed(x.astype(f32), emb_W.astype(f32), r(emb_b),
                         r(emb_g), r(emb_be), degp3, conv_W[0].astype(f32))
    srcp1 = srcp.reshape(-1, 2, K)
    dstp1 = dstp.reshape(-1, 2, K)
    G0 = G1 = None
    for l in range(4):
        Ppart = _sc_msg(hp, srcp1, dstp1)
        if l < 3:
            hp = _tc_layer(Ppart, dinv, r(conv_b[l]), r(bn_g[l]),
                           r(bn_be[l]), conv_W[l + 1].astype(f32))
        else:
            G0, G1 = _tc_final(Ppart, dinv, r(conv_b[l]), r(bn_g[l]),
                               r(bn_be[l]), mlp1_W[:H].astype(f32),
                               mlp1_W[H:2 * H].astype(f32))
    S = _sc_pair(G0, G1, i0, i1)
    y, s1, s2 = _tc_passA(S, pfp, mlp1_W[2 * H:].astype(f32), r(mlp1_b))
    u, t1, t2 = _tc_passB(y, s1, s2, r(mlp1_g), r(mlp1_be),
                          mlp2_W.astype(f32), r(mlp2_b))
    out = _tc_passC(u, t1, t2, r(mlp2_g), r(mlp2_be),
                    mlp3_W.astype(f32), mlp3_b.astype(f32).reshape(1, 1))
    return out[:P]


# speedup vs baseline: 1.0374x; 1.0010x over previous
import functools

import jax
import jax.numpy as jnp
from jax import lax
from jax.experimental import pallas as pl
from jax.experimental.pallas import tpu as pltpu
from jax.experimental.pallas import tpu_sc as plsc

N = 10000
H = 128
E = 640000
P = 200000
DP = 26
EPS = 1e-5

NC = 2
NS = 16
LANES = 16
NW = NC * NS
K = 128

N_PAD = 10112
ROWS_SUB = N_PAD // NS
PAD_NODE = 10008
CE = 160
E_PAD = NW * CE * K
DN = 10240
DSUB = DN // NS
CP = 50
P_PAD = NW * CP * K
K2 = 64
BP = 1024
GP = P_PAD // BP

_f32 = jnp.float32


def _sc_mesh():
    return plsc.VectorSubcoreMesh(core_axis_name="c", subcore_axis_name="s",
                                  num_cores=NC, num_subcores=NS)


def _make_sc_deg(interpret=False):
    @functools.partial(
        pl.kernel, mesh=_sc_mesh(), interpret=interpret,
        out_type=jax.ShapeDtypeStruct((NC, DN), _f32),
        scratch_types=[
            pltpu.VMEM((CE, K), jnp.int32),
            pltpu.VMEM((K,), _f32),
            pltpu.VMEM((DSUB,), _f32),
            pltpu.VMEM_SHARED((DN,), _f32),
            pltpu.SemaphoreType.DMA,
        ],
    )
    def deg_kernel(dst_hbm, out_hbm, didx_all, ones_v, zbuf, dacc, sem):
        c = lax.axis_index("c")
        s = lax.axis_index("s")
        wid = s * NC + c
        one16 = jnp.ones((LANES,), _f32)
        zero16 = jnp.zeros((LANES,), _f32)
        for j in range(K // LANES):
            ones_v[pl.ds(j * LANES, LANES)] = one16
        for j in range(DSUB // LANES):
            zbuf[pl.ds(j * LANES, LANES)] = zero16
        pltpu.sync_copy(zbuf, dacc.at[pl.ds(s * DSUB, DSUB)])
        pltpu.sync_copy(dst_hbm.at[wid], didx_all)
        plsc.subcore_barrier()

        FIRE = 8
        assert CE % FIRE == 0

        def body(g, carry):
            for j in range(FIRE):
                pltpu.async_copy(ones_v, dacc.at[didx_all.at[g * FIRE + j]],
                                 sem, add=True)
            for j in range(FIRE):
                pltpu.make_async_copy(
                    ones_v, dacc.at[didx_all.at[g * FIRE + j]], sem).wait()
            return carry

        lax.fori_loop(0, CE // FIRE, body, 0)
        plsc.subcore_barrier()
        pltpu.sync_copy(dacc.at[pl.ds(s * DSUB, DSUB)],
                        out_hbm.at[c, pl.ds(s * DSUB, DSUB)])

    return deg_kernel


def _make_sc_msg(interpret=False):
    @functools.partial(
        pl.kernel, mesh=_sc_mesh(), interpret=interpret,
        out_type=jax.ShapeDtypeStruct((NC, N_PAD, H), _f32),
        scratch_types=[
            pltpu.VMEM((2, K), jnp.int32),
            pltpu.VMEM((2, K), jnp.int32),
            pltpu.VMEM((2, K, H), _f32),
            pltpu.VMEM_SHARED((N_PAD, H), _f32),
            pltpu.SemaphoreType.DMA,
            pltpu.SemaphoreType.DMA,
            pltpu.SemaphoreType.DMA,
            pltpu.SemaphoreType.DMA,
        ],
    )
    def msg_kernel(hp_hbm, src_hbm, dst_hbm, out_hbm, sidx, didx,
                   rows, acc, g0, g1, s0, s1):
        c = lax.axis_index("c")
        s = lax.axis_index("s")
        wid = s * NC + c
        row0 = s * ROWS_SUB

        @pl.when(c == 0)
        def _():
            pltpu.sync_copy(hp_hbm.at[pl.ds(row0, ROWS_SUB)],
                            acc.at[pl.ds(row0, ROWS_SUB)])

        @pl.when(c != 0)
        def _():
            zero16 = jnp.zeros((LANES,), _f32)

            def zb(i, carry):
                for j in range(H // LANES):
                    rows[0, i, pl.ds(j * LANES, LANES)] = zero16
                return carry

            lax.fori_loop(0, K, zb, 0)
            nfull = ROWS_SUB // K
            rem = ROWS_SUB - nfull * K
            for t in range(nfull):
                pltpu.sync_copy(rows.at[0], acc.at[pl.ds(row0 + t * K, K)])
            pltpu.sync_copy(rows.at[0, pl.ds(0, rem)],
                            acc.at[pl.ds(row0 + nfull * K, rem)])

        plsc.subcore_barrier()

        gsems = (g0, g1)
        ssems = (s0, s1)
        NG = E_PAD // (NW * 2 * K)

        def body(g, carry):
            j = wid * NG + g
            pltpu.sync_copy(src_hbm.at[j], sidx)
            pltpu.sync_copy(dst_hbm.at[j], didx)
            ds = [pltpu.async_copy(hp_hbm.at[sidx.at[u]], rows.at[u],
                                   gsems[u]) for u in range(2)]
            ws = []
            for u in range(2):
                ds[u].wait()
                ws.append(pltpu.async_copy(rows.at[u], acc.at[didx.at[u]],
                                           ssems[u], add=True))
            for u in range(2):
                ws[u].wait()
            return carry

        lax.fori_loop(0, NG, body, 0)
        plsc.subcore_barrier()
        pltpu.sync_copy(acc.at[pl.ds(row0, ROWS_SUB)],
                        out_hbm.at[c, pl.ds(row0, ROWS_SUB)])

    return msg_kernel


def _make_sc_pair(interpret=False):
    @functools.partial(
        pl.kernel, mesh=_sc_mesh(), interpret=interpret,
        out_type=jax.ShapeDtypeStruct((P_PAD, H), _f32),
        scratch_types=[
            pltpu.VMEM((CP, K), jnp.int32),
            pltpu.VMEM((CP, K), jnp.int32),
            pltpu.VMEM((2, K, H), _f32),
            pltpu.VMEM((2, K, H), _f32),
            pltpu.SemaphoreType.DMA,
            pltpu.SemaphoreType.DMA,
            pltpu.SemaphoreType.DMA,
            pltpu.SemaphoreType.DMA,
            pltpu.SemaphoreType.DMA,
            pltpu.SemaphoreType.DMA,
        ],
    )
    def pair_kernel(g0_hbm, g1_hbm, i0_hbm, i1_hbm, s_hbm,
                    i0_all, i1_all, r0, r1,
                    g0s0, g0s1, g1s0, g1s1, ws0, ws1):
        c = lax.axis_index("c")
        s = lax.axis_index("s")
        wid = s * NC + c
        pltpu.sync_copy(i0_hbm.at[wid], i0_all)
        pltpu.sync_copy(i1_hbm.at[wid], i1_all)
        rbase = wid * (CP * K)

        def vadd(b):
            def ab(i, carry):
                for jj in range(H // LANES):
                    sl = pl.ds(jj * LANES, LANES)
                    r0[b, i, sl] = r0[b, i, sl] + r1[b, i, sl]
                return carry

            lax.fori_loop(0, K, ab, 0)

        def body(g, carry):
            c0 = g * 2
            c1 = g * 2 + 1
            d00 = pltpu.async_copy(g0_hbm.at[i0_all.at[c0]], r0.at[0], g0s0)
            d01 = pltpu.async_copy(g0_hbm.at[i0_all.at[c1]], r0.at[1], g0s1)
            d10 = pltpu.async_copy(g1_hbm.at[i1_all.at[c0]], r1.at[0], g1s0)
            d11 = pltpu.async_copy(g1_hbm.at[i1_all.at[c1]], r1.at[1], g1s1)
            d00.wait()
            d10.wait()
            vadd(0)
            w0 = pltpu.async_copy(r0.at[0],
                                  s_hbm.at[pl.ds(rbase + c0 * K, K)], ws0)
            d01.wait()
            d11.wait()
            vadd(1)
            w1 = pltpu.async_copy(r0.at[1],
                                  s_hbm.at[pl.ds(rbase + c1 * K, K)], ws1)
            w0.wait()
            w1.wait()
            return carry

        lax.fori_loop(0, CP // 2, body, 0)

    return pair_kernel


_make_sc_deg = functools.lru_cache(maxsize=None)(_make_sc_deg)
_make_sc_msg = functools.lru_cache(maxsize=None)(_make_sc_msg)
_make_sc_pair = functools.lru_cache(maxsize=None)(_make_sc_pair)


def _sc_deg(dstp):
    return _make_sc_deg()(dstp)


def _sc_msg(hp, srcp, dstp):
    return _make_sc_msg()(hp, srcp, dstp)


def _sc_pair(G0, G1, i0, i1):
    return _make_sc_pair()(G0, G1, i0, i1)


def _tc_embed_a(x, emb_W, emb_b, emb_g, emb_be, W0, interpret=False):
    def body(x_ref, w_ref, b_ref, g_ref, be_ref, w0_ref, hw_ref):
        y = jnp.dot(x_ref[...], w_ref[...],
                    preferred_element_type=_f32) + b_ref[...]
        m = jnp.mean(y, axis=0, keepdims=True)
        yc = y - m
        v = jnp.mean(yc * yc, axis=0, keepdims=True)
        h = jnp.maximum(yc * lax.rsqrt(v + EPS) * g_ref[...] + be_ref[...],
                        0.0)
        hw_ref[...] = jnp.dot(h, w0_ref[...], preferred_element_type=_f32)

    return pl.pallas_call(
        body, interpret=interpret,
        out_shape=jax.ShapeDtypeStruct((N, H), _f32),
    )(x, emb_W, emb_b, emb_g, emb_be, W0)


def _tc_scale0(degp3, hW0, interpret=False):
    def body(dp_ref, hw_ref, hp_ref, dinv_ref):
        deg = dp_ref[0] + dp_ref[1]
        dinv = lax.rsqrt(deg[:N] + 1.0)
        dinv_ref[...] = dinv
        hp_ref[:N] = dinv * hw_ref[...]
        hp_ref[N:] = jnp.zeros((N_PAD - N, H), _f32)

    return pl.pallas_call(
        body, interpret=interpret,
        out_shape=[jax.ShapeDtypeStruct((N_PAD, H), _f32),
                   jax.ShapeDtypeStruct((N, 1), _f32)],
    )(degp3, hW0)


def _tc_layer(Ppart, dinv, b, g, be, Wn, interpret=False):
    def body(p_ref, dinv_ref, b_ref, g_ref, be_ref, w_ref, out_ref):
        y = dinv_ref[...] * (p_ref[0, :N] + p_ref[1, :N]) + b_ref[...]
        m = jnp.mean(y, axis=0, keepdims=True)
        yc = y - m
        v = jnp.mean(yc * yc, axis=0, keepdims=True)
        h = jnp.maximum(yc * lax.rsqrt(v + EPS) * g_ref[...] + be_ref[...],
                        0.0)
        out_ref[:N] = dinv_ref[...] * jnp.dot(h, w_ref[...],
                                              preferred_element_type=_f32)
        out_ref[N:] = jnp.zeros((N_PAD - N, H), _f32)

    return pl.pallas_call(
        body, interpret=interpret,
        out_shape=jax.ShapeDtypeStruct((N_PAD, H), _f32),
    )(Ppart, dinv, b, g, be, Wn)


def _tc_final(Ppart, dinv, b, g, be, W1a, W1b, interpret=False):
    def body(p_ref, dinv_ref, b_ref, g_ref, be_ref, wa_ref, wb_ref,
             g0_ref, g1_ref):
        y = dinv_ref[...] * (p_ref[0, :N] + p_ref[1, :N]) + b_ref[...]
        m = jnp.mean(y, axis=0, keepdims=True)
        yc = y - m
        v = jnp.mean(yc * yc, axis=0, keepdims=True)
        h = jnp.maximum(yc * lax.rsqrt(v + EPS) * g_ref[...] + be_ref[...],
                        0.0)
        g0_ref[...] = jnp.dot(h, wa_ref[...], preferred_element_type=_f32)
        g1_ref[...] = jnp.dot(h, wb_ref[...], preferred_element_type=_f32)

    return pl.pallas_call(
        body, interpret=interpret,
        out_shape=[jax.ShapeDtypeStruct((N, H), _f32),
                   jax.ShapeDtypeStruct((N, H), _f32)],
    )(Ppart, dinv, b, g, be, W1a, W1b)


def _tc_passA(S, pfp, W1c, b1, interpret=False):
    def body(a0, pf, wc, b, y_ref, s1_ref, s2_ref):
        i = pl.program_id(0)
        y = a0[...] + jnp.dot(pf[...], wc[...],
                              preferred_element_type=_f32) + b[...]
        y_ref[...] = y
        rid = i * BP + lax.broadcasted_iota(jnp.int32, (BP, 1), 0)
        ym = jnp.where(rid < P, y, 0.0)
        ps1 = jnp.sum(ym, axis=0, keepdims=True)
        ps2 = jnp.sum(ym * ym, axis=0, keepdims=True)

        @pl.when(i == 0)
        def _():
            s1_ref[...] = ps1
            s2_ref[...] = ps2

        @pl.when(i != 0)
        def _():
            s1_ref[...] = s1_ref[...] + ps1
            s2_ref[...] = s2_ref[...] + ps2

    return pl.pallas_call(
        body, interpret=interpret, grid=(GP,),
        in_specs=[pl.BlockSpec((BP, H), lambda i: (i, 0)),
                  pl.BlockSpec((BP, DP), lambda i: (i, 0)),
                  pl.BlockSpec((DP, H), lambda i: (0, 0)),
                  pl.BlockSpec((1, H), lambda i: (0, 0))],
        out_specs=[pl.BlockSpec((BP, H), lambda i: (i, 0)),
                   pl.BlockSpec((1, H), lambda i: (0, 0)),
                   pl.BlockSpec((1, H), lambda i: (0, 0))],
        out_shape=[jax.ShapeDtypeStruct((P_PAD, H), _f32),
                   jax.ShapeDtypeStruct((1, H), _f32),
                   jax.ShapeDtypeStruct((1, H), _f32)],
    )(S, pfp, W1c, b1)


def _tc_passB(y, s1, s2, g1, be1, W2, b2, interpret=False):
    H2 = H // 2

    def body(y_ref, s1r, s2r, gr, ber, wr, br, u_ref, t1_ref, t2_ref):
        i = pl.program_id(0)
        m = s1r[...] * (1.0 / P)
        var = s2r[...] * (1.0 / P) - m * m
        z = jnp.maximum((y_ref[...] - m) * lax.rsqrt(var + EPS) * gr[...]
                        + ber[...], 0.0)
        u = jnp.dot(z, wr[...], preferred_element_type=_f32) + br[...]
        u_ref[...] = u
        rid = i * BP + lax.broadcasted_iota(jnp.int32, (BP, 1), 0)
        um = jnp.where(rid < P, u, 0.0)
        ps1 = jnp.sum(um, axis=0, keepdims=True)
        ps2 = jnp.sum(um * um, axis=0, keepdims=True)

        @pl.when(i == 0)
        def _():
            t1_ref[...] = ps1
            t2_ref[...] = ps2

        @pl.when(i != 0)
        def _():
            t1_ref[...] = t1_ref[...] + ps1
            t2_ref[...] = t2_ref[...] + ps2

    return pl.pallas_call(
        body, interpret=interpret, grid=(GP,),
        in_specs=[pl.BlockSpec((BP, H), lambda i: (i, 0)),
                  pl.BlockSpec((1, H), lambda i: (0, 0)),
                  pl.BlockSpec((1, H), lambda i: (0, 0)),
                  pl.BlockSpec((1, H), lambda i: (0, 0)),
                  pl.BlockSpec((1, H), lambda i: (0, 0)),
                  pl.BlockSpec((H, H2), lambda i: (0, 0)),
                  pl.BlockSpec((1, H2), lambda i: (0, 0))],
        out_specs=[pl.BlockSpec((BP, H2), lambda i: (i, 0)),
                   pl.BlockSpec((1, H2), lambda i: (0, 0)),
                   pl.BlockSpec((1, H2), lambda i: (0, 0))],
        out_shape=[jax.ShapeDtypeStruct((P_PAD, H2), _f32),
                   jax.ShapeDtypeStruct((1, H2), _f32),
                   jax.ShapeDtypeStruct((1, H2), _f32)],
    )(y, s1, s2, g1, be1, W2, b2)


def _tc_passC(u, t1, t2, g2, be2, W3, b3, interpret=False):
    H2 = H // 2

    def body(u_ref, t1r, t2r, gr, ber, wr, br, o_ref):
        m = t1r[...] * (1.0 / P)
        var = t2r[...] * (1.0 / P) - m * m
        z = jnp.maximum((u_ref[...] - m) * lax.rsqrt(var + EPS) * gr[...]
                        + ber[...], 0.0)
        o_ref[...] = jnp.dot(z, wr[...], preferred_element_type=_f32) + br[...]

    return pl.pallas_call(
        body, interpret=interpret, grid=(GP,),
        in_specs=[pl.BlockSpec((BP, H2), lambda i: (i, 0)),
                  pl.BlockSpec((1, H2), lambda i: (0, 0)),
                  pl.BlockSpec((1, H2), lambda i: (0, 0)),
                  pl.BlockSpec((1, H2), lambda i: (0, 0)),
                  pl.BlockSpec((1, H2), lambda i: (0, 0)),
                  pl.BlockSpec((H2, 1), lambda i: (0, 0)),
                  pl.BlockSpec((1, 1), lambda i: (0, 0))],
        out_specs=pl.BlockSpec((BP, 1), lambda i: (i, 0)),
        out_shape=jax.ShapeDtypeStruct((P_PAD, 1), _f32),
    )(u, t1, t2, g2, be2, W3, b3)


def kernel(x, edge_index, batch, pair_indices, pair_features,
           emb_W, emb_b, emb_g, emb_be,
           conv_W, conv_b, bn_g, bn_be,
           mlp1_W, mlp1_b, mlp1_g, mlp1_be,
           mlp2_W, mlp2_b, mlp2_g, mlp2_be,
           mlp3_W, mlp3_b):
    f32 = _f32
    epad = N + (jnp.arange(E_PAD - E, dtype=jnp.int32) % (N_PAD - N))
    srcp = jnp.concatenate([edge_index[0].astype(jnp.int32),
                            epad]).reshape(NW, CE, K)
    dstp = jnp.concatenate([edge_index[1].astype(jnp.int32),
                            epad]).reshape(NW, CE, K)
    ppad = jnp.arange(P_PAD - P, dtype=jnp.int32) % N
    i0 = jnp.concatenate([pair_indices[:, 0].astype(jnp.int32),
                          ppad]).reshape(NW, CP, K)
    i1 = jnp.concatenate([pair_indices[:, 1].astype(jnp.int32),
                          ppad]).reshape(NW, CP, K)
    pfp = jnp.concatenate(
        [pair_features.astype(f32), jnp.zeros((P_PAD - P, DP), f32)], axis=0)

    def r(a):
        return a.astype(f32).reshape(1, -1)

    degp = _sc_deg(dstp)
    degp3 = degp.reshape(NC, DN, 1)
    hW0 = _tc_embed_a(x.astype(f32), emb_W.astype(f32), r(emb_b),
                      r(emb_g), r(emb_be), conv_W[0].astype(f32))
    hp, dinv = _tc_scale0(degp3, hW0)
    srcp1 = srcp.reshape(-1, 2, K)
    dstp1 = dstp.reshape(-1, 2, K)
    G0 = G1 = None
    for l in range(4):
        Ppart = _sc_msg(hp, srcp1, dstp1)
        if l < 3:
            hp = _tc_layer(Ppart, dinv, r(conv_b[l]), r(bn_g[l]),
                           r(bn_be[l]), conv_W[l + 1].astype(f32))
        else:
            G0, G1 = _tc_final(Ppart, dinv, r(conv_b[l]), r(bn_g[l]),
                               r(bn_be[l]), mlp1_W[:H].astype(f32),
                               mlp1_W[H:2 * H].astype(f32))
    S = _sc_pair(G0, G1, i0, i1)
    y, s1, s2 = _tc_passA(S, pfp, mlp1_W[2 * H:].astype(f32), r(mlp1_b))
    u, t1, t2 = _tc_passB(y, s1, s2, r(mlp1_g), r(mlp1_be),
                          mlp2_W.astype(f32), r(mlp2_b))
    out = _tc_passC(u, t1, t2, r(mlp2_g), r(mlp2_be),
                    mlp3_W.astype(f32), mlp3_b.astype(f32).reshape(1, 1))
    return out[:P]


# speedup vs baseline: 1.1415x; 1.1003x over previous
import functools

import jax
import jax.numpy as jnp
from jax import lax
from jax.experimental import pallas as pl
from jax.experimental.pallas import tpu as pltpu
from jax.experimental.pallas import tpu_sc as plsc

N = 10000
H = 128
E = 640000
P = 200000
DP = 26
EPS = 1e-5

NC = 2
NS = 16
LANES = 16
NW = NC * NS
K = 128

N_PAD = 10112
ROWS_SUB = N_PAD // NS
PAD_NODE = 10008
CE = 160
E_PAD = NW * CE * K
DN = 10240
DSUB = DN // NS
CP = 50
P_PAD = NW * CP * K
K2 = 64
BP = 1024
GP = P_PAD // BP

_f32 = jnp.float32


def _sc_mesh():
    return plsc.VectorSubcoreMesh(core_axis_name="c", subcore_axis_name="s",
                                  num_cores=NC, num_subcores=NS)


def _make_sc_deg(interpret=False):
    @functools.partial(
        pl.kernel, mesh=_sc_mesh(), interpret=interpret,
        out_type=jax.ShapeDtypeStruct((NC, DN), _f32),
        scratch_types=[
            pltpu.VMEM((CE, K), jnp.int32),
            pltpu.VMEM((K,), _f32),
            pltpu.VMEM((DSUB,), _f32),
            pltpu.VMEM_SHARED((DN,), _f32),
            pltpu.SemaphoreType.DMA,
        ],
    )
    def deg_kernel(dst_hbm, out_hbm, didx_all, ones_v, zbuf, dacc, sem):
        c = lax.axis_index("c")
        s = lax.axis_index("s")
        wid = s * NC + c
        one16 = jnp.ones((LANES,), _f32)
        zero16 = jnp.zeros((LANES,), _f32)
        for j in range(K // LANES):
            ones_v[pl.ds(j * LANES, LANES)] = one16
        for j in range(DSUB // LANES):
            zbuf[pl.ds(j * LANES, LANES)] = zero16
        pltpu.sync_copy(zbuf, dacc.at[pl.ds(s * DSUB, DSUB)])
        pltpu.sync_copy(dst_hbm.at[wid], didx_all)
        plsc.subcore_barrier()

        FIRE = 8
        assert CE % FIRE == 0

        def body(g, carry):
            for j in range(FIRE):
                pltpu.async_copy(ones_v, dacc.at[didx_all.at[g * FIRE + j]],
                                 sem, add=True)
            for j in range(FIRE):
                pltpu.make_async_copy(
                    ones_v, dacc.at[didx_all.at[g * FIRE + j]], sem).wait()
            return carry

        lax.fori_loop(0, CE // FIRE, body, 0)
        plsc.subcore_barrier()
        pltpu.sync_copy(dacc.at[pl.ds(s * DSUB, DSUB)],
                        out_hbm.at[c, pl.ds(s * DSUB, DSUB)])

    return deg_kernel


def _make_sc_msg(interpret=False):
    @functools.partial(
        pl.kernel, mesh=_sc_mesh(), interpret=interpret,
        out_type=jax.ShapeDtypeStruct((NC, N_PAD, H), _f32),
        scratch_types=[
            pltpu.VMEM((8, K), jnp.int32),
            pltpu.VMEM((8, K), jnp.int32),
            pltpu.VMEM((2, K, H), _f32),
            pltpu.VMEM_SHARED((N_PAD, H), _f32),
            pltpu.SemaphoreType.DMA,
            pltpu.SemaphoreType.DMA,
            pltpu.SemaphoreType.DMA,
            pltpu.SemaphoreType.DMA,
        ],
    )
    def msg_kernel(hp_hbm, src_hbm, dst_hbm, out_hbm, sidx, didx,
                   rows, acc, g0, g1, s0, s1):
        c = lax.axis_index("c")
        s = lax.axis_index("s")
        wid = s * NC + c
        row0 = s * ROWS_SUB

        @pl.when(c == 0)
        def _():
            pltpu.sync_copy(hp_hbm.at[pl.ds(row0, ROWS_SUB)],
                            acc.at[pl.ds(row0, ROWS_SUB)])

        @pl.when(c != 0)
        def _():
            zero16 = jnp.zeros((LANES,), _f32)

            def zb(i, carry):
                for j in range(H // LANES):
                    rows[0, i, pl.ds(j * LANES, LANES)] = zero16
                return carry

            lax.fori_loop(0, K, zb, 0)
            nfull = ROWS_SUB // K
            rem = ROWS_SUB - nfull * K
            for t in range(nfull):
                pltpu.sync_copy(rows.at[0], acc.at[pl.ds(row0 + t * K, K)])
            pltpu.sync_copy(rows.at[0, pl.ds(0, rem)],
                            acc.at[pl.ds(row0 + nfull * K, rem)])

        plsc.subcore_barrier()

        gsems = (g0, g1)
        ssems = (s0, s1)
        NG = E_PAD // (NW * 8 * K)

        def body(g, carry):
            j = wid * NG + g
            pltpu.sync_copy(src_hbm.at[j], sidx)
            pltpu.sync_copy(dst_hbm.at[j], didx)
            for q in range(4):
                u0, u1 = 2 * q, 2 * q + 1
                d0 = pltpu.async_copy(hp_hbm.at[sidx.at[u0]], rows.at[0],
                                      gsems[0])
                d1 = pltpu.async_copy(hp_hbm.at[sidx.at[u1]], rows.at[1],
                                      gsems[1])
                d0.wait()
                w0 = pltpu.async_copy(rows.at[0], acc.at[didx.at[u0]],
                                      ssems[0], add=True)
                d1.wait()
                w1 = pltpu.async_copy(rows.at[1], acc.at[didx.at[u1]],
                                      ssems[1], add=True)
                w0.wait()
                w1.wait()
            return carry

        lax.fori_loop(0, NG, body, 0)
        plsc.subcore_barrier()
        pltpu.sync_copy(acc.at[pl.ds(row0, ROWS_SUB)],
                        out_hbm.at[c, pl.ds(row0, ROWS_SUB)])

    return msg_kernel


def _make_sc_pair(interpret=False):
    @functools.partial(
        pl.kernel, mesh=_sc_mesh(), interpret=interpret,
        out_type=jax.ShapeDtypeStruct((P_PAD, H), _f32),
        scratch_types=[
            pltpu.VMEM((CP, K), jnp.int32),
            pltpu.VMEM((CP, K), jnp.int32),
            pltpu.VMEM((2, K, H), _f32),
            pltpu.VMEM((2, K, H), _f32),
            pltpu.SemaphoreType.DMA,
            pltpu.SemaphoreType.DMA,
            pltpu.SemaphoreType.DMA,
            pltpu.SemaphoreType.DMA,
            pltpu.SemaphoreType.DMA,
            pltpu.SemaphoreType.DMA,
        ],
    )
    def pair_kernel(g0_hbm, g1_hbm, i0_hbm, i1_hbm, s_hbm,
                    i0_all, i1_all, r0, r1,
                    g0s0, g0s1, g1s0, g1s1, ws0, ws1):
        c = lax.axis_index("c")
        s = lax.axis_index("s")
        wid = s * NC + c
        pltpu.sync_copy(i0_hbm.at[wid], i0_all)
        pltpu.sync_copy(i1_hbm.at[wid], i1_all)
        rbase = wid * (CP * K)

        def vadd(b):
            def ab(i, carry):
                for jj in range(H // LANES):
                    sl = pl.ds(jj * LANES, LANES)
                    r0[b, i, sl] = r0[b, i, sl] + r1[b, i, sl]
                return carry

            lax.fori_loop(0, K, ab, 0)

        def body(g, carry):
            c0 = g * 2
            c1 = g * 2 + 1
            d00 = pltpu.async_copy(g0_hbm.at[i0_all.at[c0]], r0.at[0], g0s0)
            d01 = pltpu.async_copy(g0_hbm.at[i0_all.at[c1]], r0.at[1], g0s1)
            d10 = pltpu.async_copy(g1_hbm.at[i1_all.at[c0]], r1.at[0], g1s0)
            d11 = pltpu.async_copy(g1_hbm.at[i1_all.at[c1]], r1.at[1], g1s1)
            d00.wait()
            d10.wait()
            vadd(0)
            w0 = pltpu.async_copy(r0.at[0],
                                  s_hbm.at[pl.ds(rbase + c0 * K, K)], ws0)
            d01.wait()
            d11.wait()
            vadd(1)
            w1 = pltpu.async_copy(r0.at[1],
                                  s_hbm.at[pl.ds(rbase + c1 * K, K)], ws1)
            w0.wait()
            w1.wait()
            return carry

        lax.fori_loop(0, CP // 2, body, 0)

    return pair_kernel


_make_sc_deg = functools.lru_cache(maxsize=None)(_make_sc_deg)
_make_sc_msg = functools.lru_cache(maxsize=None)(_make_sc_msg)
_make_sc_pair = functools.lru_cache(maxsize=None)(_make_sc_pair)


def _sc_deg(dstp):
    return _make_sc_deg()(dstp)


def _sc_msg(hp, srcp, dstp):
    return _make_sc_msg()(hp, srcp, dstp)


def _sc_pair(G0, G1, i0, i1):
    return _make_sc_pair()(G0, G1, i0, i1)


def _tc_embed_a(x, emb_W, emb_b, emb_g, emb_be, W0, interpret=False):
    def body(x_ref, w_ref, b_ref, g_ref, be_ref, w0_ref, hw_ref):
        y = jnp.dot(x_ref[...], w_ref[...],
                    preferred_element_type=_f32) + b_ref[...]
        m = jnp.mean(y, axis=0, keepdims=True)
        yc = y - m
        v = jnp.mean(yc * yc, axis=0, keepdims=True)
        h = jnp.maximum(yc * lax.rsqrt(v + EPS) * g_ref[...] + be_ref[...],
                        0.0)
        hw_ref[...] = jnp.dot(h, w0_ref[...], preferred_element_type=_f32)

    return pl.pallas_call(
        body, interpret=interpret,
        out_shape=jax.ShapeDtypeStruct((N, H), _f32),
    )(x, emb_W, emb_b, emb_g, emb_be, W0)


def _tc_scale0(degp3, hW0, interpret=False):
    def body(dp_ref, hw_ref, hp_ref, dinv_ref):
        deg = dp_ref[0] + dp_ref[1]
        dinv = lax.rsqrt(deg[:N] + 1.0)
        dinv_ref[...] = dinv
        hp_ref[:N] = dinv * hw_ref[...]
        hp_ref[N:] = jnp.zeros((N_PAD - N, H), _f32)

    return pl.pallas_call(
        body, interpret=interpret,
        out_shape=[jax.ShapeDtypeStruct((N_PAD, H), _f32),
                   jax.ShapeDtypeStruct((N, 1), _f32)],
    )(degp3, hW0)


def _tc_layer(Ppart, dinv, b, g, be, Wn, interpret=False):
    def body(p_ref, dinv_ref, b_ref, g_ref, be_ref, w_ref, out_ref):
        y = dinv_ref[...] * (p_ref[0, :N] + p_ref[1, :N]) + b_ref[...]
        m = jnp.mean(y, axis=0, keepdims=True)
        yc = y - m
        v = jnp.mean(yc * yc, axis=0, keepdims=True)
        h = jnp.maximum(yc * lax.rsqrt(v + EPS) * g_ref[...] + be_ref[...],
                        0.0)
        out_ref[:N] = dinv_ref[...] * jnp.dot(h, w_ref[...],
                                              preferred_element_type=_f32)
        out_ref[N:] = jnp.zeros((N_PAD - N, H), _f32)

    return pl.pallas_call(
        body, interpret=interpret,
        out_shape=jax.ShapeDtypeStruct((N_PAD, H), _f32),
    )(Ppart, dinv, b, g, be, Wn)


def _tc_final(Ppart, dinv, b, g, be, W1a, W1b, interpret=False):
    def body(p_ref, dinv_ref, b_ref, g_ref, be_ref, wa_ref, wb_ref,
             g0_ref, g1_ref):
        y = dinv_ref[...] * (p_ref[0, :N] + p_ref[1, :N]) + b_ref[...]
        m = jnp.mean(y, axis=0, keepdims=True)
        yc = y - m
        v = jnp.mean(yc * yc, axis=0, keepdims=True)
        h = jnp.maximum(yc * lax.rsqrt(v + EPS) * g_ref[...] + be_ref[...],
                        0.0)
        g0_ref[...] = jnp.dot(h, wa_ref[...], preferred_element_type=_f32)
        g1_ref[...] = jnp.dot(h, wb_ref[...], preferred_element_type=_f32)

    return pl.pallas_call(
        body, interpret=interpret,
        out_shape=[jax.ShapeDtypeStruct((N, H), _f32),
                   jax.ShapeDtypeStruct((N, H), _f32)],
    )(Ppart, dinv, b, g, be, W1a, W1b)


def _tc_passA(S, pfp, W1c, b1, interpret=False):
    def body(a0, pf, wc, b, y_ref, s1_ref, s2_ref):
        i = pl.program_id(0)
        y = a0[...] + jnp.dot(pf[...], wc[...],
                              preferred_element_type=_f32) + b[...]
        y_ref[...] = y
        rid = i * BP + lax.broadcasted_iota(jnp.int32, (BP, 1), 0)
        ym = jnp.where(rid < P, y, 0.0)
        ps1 = jnp.sum(ym, axis=0, keepdims=True)
        ps2 = jnp.sum(ym * ym, axis=0, keepdims=True)

        @pl.when(i == 0)
        def _():
            s1_ref[...] = ps1
            s2_ref[...] = ps2

        @pl.when(i != 0)
        def _():
            s1_ref[...] = s1_ref[...] + ps1
            s2_ref[...] = s2_ref[...] + ps2

    return pl.pallas_call(
        body, interpret=interpret, grid=(GP,),
        in_specs=[pl.BlockSpec((BP, H), lambda i: (i, 0)),
                  pl.BlockSpec((BP, DP), lambda i: (i, 0)),
                  pl.BlockSpec((DP, H), lambda i: (0, 0)),
                  pl.BlockSpec((1, H), lambda i: (0, 0))],
        out_specs=[pl.BlockSpec((BP, H), lambda i: (i, 0)),
                   pl.BlockSpec((1, H), lambda i: (0, 0)),
                   pl.BlockSpec((1, H), lambda i: (0, 0))],
        out_shape=[jax.ShapeDtypeStruct((P_PAD, H), _f32),
                   jax.ShapeDtypeStruct((1, H), _f32),
                   jax.ShapeDtypeStruct((1, H), _f32)],
    )(S, pfp, W1c, b1)


def _tc_passB(y, s1, s2, g1, be1, W2, b2, interpret=False):
    H2 = H // 2

    def body(y_ref, s1r, s2r, gr, ber, wr, br, u_ref, t1_ref, t2_ref):
        i = pl.program_id(0)
        m = s1r[...] * (1.0 / P)
        var = s2r[...] * (1.0 / P) - m * m
        z = jnp.maximum((y_ref[...] - m) * lax.rsqrt(var + EPS) * gr[...]
                        + ber[...], 0.0)
        u = jnp.dot(z, wr[...], preferred_element_type=_f32) + br[...]
        u_ref[...] = u
        rid = i * BP + lax.broadcasted_iota(jnp.int32, (BP, 1), 0)
        um = jnp.where(rid < P, u, 0.0)
        ps1 = jnp.sum(um, axis=0, keepdims=True)
        ps2 = jnp.sum(um * um, axis=0, keepdims=True)

        @pl.when(i == 0)
        def _():
            t1_ref[...] = ps1
            t2_ref[...] = ps2

        @pl.when(i != 0)
        def _():
            t1_ref[...] = t1_ref[...] + ps1
            t2_ref[...] = t2_ref[...] + ps2

    return pl.pallas_call(
        body, interpret=interpret, grid=(GP,),
        in_specs=[pl.BlockSpec((BP, H), lambda i: (i, 0)),
                  pl.BlockSpec((1, H), lambda i: (0, 0)),
                  pl.BlockSpec((1, H), lambda i: (0, 0)),
                  pl.BlockSpec((1, H), lambda i: (0, 0)),
                  pl.BlockSpec((1, H), lambda i: (0, 0)),
                  pl.BlockSpec((H, H2), lambda i: (0, 0)),
                  pl.BlockSpec((1, H2), lambda i: (0, 0))],
        out_specs=[pl.BlockSpec((BP, H2), lambda i: (i, 0)),
                   pl.BlockSpec((1, H2), lambda i: (0, 0)),
                   pl.BlockSpec((1, H2), lambda i: (0, 0))],
        out_shape=[jax.ShapeDtypeStruct((P_PAD, H2), _f32),
                   jax.ShapeDtypeStruct((1, H2), _f32),
                   jax.ShapeDtypeStruct((1, H2), _f32)],
    )(y, s1, s2, g1, be1, W2, b2)


def _tc_passC(u, t1, t2, g2, be2, W3, b3, interpret=False):
    H2 = H // 2

    def body(u_ref, t1r, t2r, gr, ber, wr, br, o_ref):
        m = t1r[...] * (1.0 / P)
        var = t2r[...] * (1.0 / P) - m * m
        z = jnp.maximum((u_ref[...] - m) * lax.rsqrt(var + EPS) * gr[...]
                        + ber[...], 0.0)
        o_ref[...] = jnp.dot(z, wr[...], preferred_element_type=_f32) + br[...]

    return pl.pallas_call(
        body, interpret=interpret, grid=(GP,),
        in_specs=[pl.BlockSpec((BP, H2), lambda i: (i, 0)),
                  pl.BlockSpec((1, H2), lambda i: (0, 0)),
                  pl.BlockSpec((1, H2), lambda i: (0, 0)),
                  pl.BlockSpec((1, H2), lambda i: (0, 0)),
                  pl.BlockSpec((1, H2), lambda i: (0, 0)),
                  pl.BlockSpec((H2, 1), lambda i: (0, 0)),
                  pl.BlockSpec((1, 1), lambda i: (0, 0))],
        out_specs=pl.BlockSpec((BP, 1), lambda i: (i, 0)),
        out_shape=jax.ShapeDtypeStruct((P_PAD, 1), _f32),
    )(u, t1, t2, g2, be2, W3, b3)


def kernel(x, edge_index, batch, pair_indices, pair_features,
           emb_W, emb_b, emb_g, emb_be,
           conv_W, conv_b, bn_g, bn_be,
           mlp1_W, mlp1_b, mlp1_g, mlp1_be,
           mlp2_W, mlp2_b, mlp2_g, mlp2_be,
           mlp3_W, mlp3_b):
    f32 = _f32
    epad = N + (jnp.arange(E_PAD - E, dtype=jnp.int32) % (N_PAD - N))
    srcp = jnp.concatenate([edge_index[0].astype(jnp.int32),
                            epad]).reshape(NW, CE, K)
    dstp = jnp.concatenate([edge_index[1].astype(jnp.int32),
                            epad]).reshape(NW, CE, K)
    ppad = jnp.arange(P_PAD - P, dtype=jnp.int32) % N
    i0 = jnp.concatenate([pair_indices[:, 0].astype(jnp.int32),
                          ppad]).reshape(NW, CP, K)
    i1 = jnp.concatenate([pair_indices[:, 1].astype(jnp.int32),
                          ppad]).reshape(NW, CP, K)
    pfp = jnp.concatenate(
        [pair_features.astype(f32), jnp.zeros((P_PAD - P, DP), f32)], axis=0)

    def r(a):
        return a.astype(f32).reshape(1, -1)

    degp = _sc_deg(dstp)
    degp3 = degp.reshape(NC, DN, 1)
    hW0 = _tc_embed_a(x.astype(f32), emb_W.astype(f32), r(emb_b),
                      r(emb_g), r(emb_be), conv_W[0].astype(f32))
    hp, dinv = _tc_scale0(degp3, hW0)
    srcp1 = srcp.reshape(-1, 8, K)
    dstp1 = dstp.reshape(-1, 8, K)
    G0 = G1 = None
    for l in range(4):
        Ppart = _sc_msg(hp, srcp1, dstp1)
        if l < 3:
            hp = _tc_layer(Ppart, dinv, r(conv_b[l]), r(bn_g[l]),
                           r(bn_be[l]), conv_W[l + 1].astype(f32))
        else:
            G0, G1 = _tc_final(Ppart, dinv, r(conv_b[l]), r(bn_g[l]),
                               r(bn_be[l]), mlp1_W[:H].astype(f32),
                               mlp1_W[H:2 * H].astype(f32))
    S = _sc_pair(G0, G1, i0, i1)
    y, s1, s2 = _tc_passA(S, pfp, mlp1_W[2 * H:].astype(f32), r(mlp1_b))
    u, t1, t2 = _tc_passB(y, s1, s2, r(mlp1_g), r(mlp1_be),
                          mlp2_W.astype(f32), r(mlp2_b))
    out = _tc_passC(u, t1, t2, r(mlp2_g), r(mlp2_be),
                    mlp3_W.astype(f32), mlp3_b.astype(f32).reshape(1, 1))
    return out[:P]
